# Initial kernel scaffold; baseline (speedup 1.0000x reference)
import jax
import jax.numpy as jnp
from jax.experimental import pallas as pl


def kernel(x, Wc, bc):
    raise NotImplementedError("write your pallas kernel here")


# speedup vs baseline: 5.9218x; 5.9218x over previous
import functools

import jax
import jax.numpy as jnp
from jax import lax
from jax.experimental import pallas as pl
from jax.experimental.pallas import tpu as pltpu
from jax.experimental.pallas import tpu_sc as plsc

B, C, H, W = 2, 192, 224, 224
HW = H * W
NROWS = B * HW
NW = 32
RPW = NROWS // NW
CHUNK = 64
NCHUNK = RPW // CHUNK
HWB = HW // 8
NV = C // 16


def _tc_index_kernel(wf_ref, x_ref, bc_ref, idx_ref, g_ref, u_ref):
    i = pl.program_id(1)
    u_ref[:, pl.ds(i * HWB, HWB)] = jnp.dot(
        wf_ref[...], x_ref[0], preferred_element_type=jnp.float32)

    @pl.when(i == (HW // HWB) - 1)
    def _():
        b = pl.program_id(0)
        u = u_ref[...]
        wcol = lax.broadcasted_iota(jnp.int32, (2, HW), 1) % W
        off = jnp.zeros((2, HW), jnp.float32)
        for t in range(9):
            dy, dx = t // 3, t % 3
            s = (dy - 1) * W + (dx - 1)
            ut = jnp.concatenate([u[t:t + 1, :], u[9 + t:10 + t, :]], axis=0)
            if s > 0:
                sh = jnp.concatenate(
                    [ut[:, s:], jnp.zeros((2, s), jnp.float32)], axis=1)
            elif s < 0:
                sh = jnp.concatenate(
                    [jnp.zeros((2, -s), jnp.float32), ut[:, :s]], axis=1)
            else:
                sh = ut
            if dx == 0:
                sh = jnp.where(wcol != 0, sh, 0.0)
            elif dx == 2:
                sh = jnp.where(wcol != W - 1, sh, 0.0)
            off = off + sh
        pos = lax.broadcasted_iota(jnp.int32, (1, HW), 1)
        gx = (pos // W).astype(jnp.float32)
        gy = (pos % W).astype(jnp.float32)
        px = jnp.clip(gx + off[0:1, :] + bc_ref[0], 0.0, float(H - 2))
        py = jnp.clip(gy + off[1:2, :] + bc_ref[1], 0.0, float(W - 2))
        qx = jnp.floor(px)
        qy = jnp.floor(py)
        fx = px - qx
        fy = py - qy
        base = b * HW + qx.astype(jnp.int32) * W + qy.astype(jnp.int32)
        idx_ref[0] = jnp.concatenate(
            [base, base + (W + 1), base + 1, base + W], axis=0)
        omx = 1.0 - fx
        omy = 1.0 - fy
        g_ref[0] = jnp.concatenate(
            [omx * omy, fx * fy, omx * fy, fx * omy], axis=0)


def _tc_index(x_flat, wflat, bc):
    return pl.pallas_call(
        _tc_index_kernel,
        grid=(B, HW // HWB),
        in_specs=[
            pl.BlockSpec((18, C), lambda b, i: (0, 0)),
            pl.BlockSpec((1, C, HWB), lambda b, i: (b, 0, i)),
            pl.BlockSpec(memory_space=pltpu.SMEM),
        ],
        out_specs=[
            pl.BlockSpec((1, 4, HW), lambda b, i: (b, 0, 0)),
            pl.BlockSpec((1, 4, HW), lambda b, i: (b, 0, 0)),
        ],
        out_shape=[
            jax.ShapeDtypeStruct((B, 4, HW), jnp.int32),
            jax.ShapeDtypeStruct((B, 4, HW), jnp.float32),
        ],
        scratch_shapes=[pltpu.VMEM((18, HW), jnp.float32)],
    )(wflat, x_flat, bc)


def _sc_gather_combine(xt, idx, g):
    mesh = plsc.VectorSubcoreMesh(core_axis_name="c", subcore_axis_name="s")

    @functools.partial(
        pl.kernel,
        mesh=mesh,
        out_type=jax.ShapeDtypeStruct((NROWS, C), jnp.float32),
        compiler_params=pltpu.CompilerParams(use_tc_tiling_on_sc=False),
        scratch_types=[
            pltpu.VMEM((RPW,), jnp.float32),
            pltpu.VMEM((RPW,), jnp.float32),
            pltpu.VMEM((RPW,), jnp.float32),
            pltpu.VMEM((RPW,), jnp.float32),
            pltpu.VMEM((CHUNK,), jnp.int32),
            pltpu.VMEM((CHUNK,), jnp.int32),
            pltpu.VMEM((CHUNK,), jnp.int32),
            pltpu.VMEM((CHUNK,), jnp.int32),
            pltpu.VMEM((CHUNK, C), jnp.float32),
            pltpu.VMEM((CHUNK, C), jnp.float32),
            pltpu.VMEM((CHUNK, C), jnp.float32),
            pltpu.VMEM((CHUNK, C), jnp.float32),
            pltpu.VMEM((CHUNK, C), jnp.float32),
            pltpu.SemaphoreType.DMA,
        ],
    )
    def sc_kernel(idx_hbm, g_hbm, xt_hbm, out_hbm,
                  gw0, gw1, gw2, gw3, ic0, ic1, ic2, ic3,
                  b0, b1, b2, b3, ob, sem):
        wid = lax.axis_index("s") * 2 + lax.axis_index("c")
        batch = wid // (HW // RPW)
        inb = (wid % (HW // RPW)) * RPW
        gws = (gw0, gw1, gw2, gw3)
        ics = (ic0, ic1, ic2, ic3)
        for k in range(4):
            pltpu.sync_copy(g_hbm.at[pl.ds((batch * 4 + k) * HW + inb, RPW)],
                            gws[k])

        @pl.loop(0, NCHUNK)
        def _chunk(j):
            off = j * CHUNK
            for k in range(4):
                pltpu.sync_copy(
                    idx_hbm.at[pl.ds((batch * 4 + k) * HW + inb + off, CHUNK)],
                    ics[k])
            bufs = (b0, b1, b2, b3)
            copies = [
                pltpu.async_copy(xt_hbm.at[ics[k]], bufs[k], sem)
                for k in range(4)
            ]
            for cp in copies:
                cp.wait()

            @pl.loop(0, CHUNK, step=16)
            def _grp(r0):
                gv0 = gw0[pl.ds(off + r0, 16)]
                gv1 = gw1[pl.ds(off + r0, 16)]
                gv2 = gw2[pl.ds(off + r0, 16)]
                gv3 = gw3[pl.ds(off + r0, 16)]
                for rr in range(16):
                    r = r0 + rr
                    g0, g1, g2, g3 = gv0[rr], gv1[rr], gv2[rr], gv3[rr]
                    for v in range(NV):
                        sl = pl.ds(v * 16, 16)
                        ob[r, sl] = (g0 * b0[r, sl] + g1 * b1[r, sl]
                                     + g2 * b2[r, sl] + g3 * b3[r, sl])

            pltpu.sync_copy(ob, out_hbm.at[pl.ds(wid * RPW + off, CHUNK)])

    return sc_kernel(idx.reshape(-1), g.reshape(-1), xt)


def kernel(x, Wc, bc):
    x_flat = x.reshape(B, C, HW)
    wflat = Wc.transpose(0, 2, 3, 1).reshape(2 * 9, C)
    idx, g = _tc_index(x_flat, wflat, bc)
    xt = x_flat.transpose(0, 2, 1).reshape(NROWS, C)
    out_rows = _sc_gather_combine(xt, idx, g)
    return out_rows.reshape(B, HW, C).transpose(0, 2, 1).reshape(B, C, H, W, 1)


# speedup vs baseline: 6.0271x; 1.0178x over previous
import functools

import jax
import jax.numpy as jnp
from jax import lax
from jax.experimental import pallas as pl
from jax.experimental.pallas import tpu as pltpu
from jax.experimental.pallas import tpu_sc as plsc

B, C, H, W = 2, 192, 224, 224
HW = H * W
NROWS = B * HW
NW = 32
RPW = NROWS // NW
CHUNK = 64
NCHUNK = RPW // CHUNK
HWB = HW // 8
NV = C // 16


def _tc_index_kernel(wf_ref, x_ref, bc_ref, idx_ref, g_ref, xt_ref, u_ref):
    i = pl.program_id(1)
    xb = x_ref[0]
    u_ref[:, pl.ds(i * HWB, HWB)] = jnp.dot(
        wf_ref[...], xb, preferred_element_type=jnp.float32)
    xt_ref[...] = xb.T

    @pl.when(i == (HW // HWB) - 1)
    def _():
        b = pl.program_id(0)
        u = u_ref[...]
        wcol = lax.broadcasted_iota(jnp.int32, (2, HW), 1) % W
        off = jnp.zeros((2, HW), jnp.float32)
        for t in range(9):
            dy, dx = t // 3, t % 3
            s = (dy - 1) * W + (dx - 1)
            ut = jnp.concatenate([u[t:t + 1, :], u[9 + t:10 + t, :]], axis=0)
            if s > 0:
                sh = jnp.concatenate(
                    [ut[:, s:], jnp.zeros((2, s), jnp.float32)], axis=1)
            elif s < 0:
                sh = jnp.concatenate(
                    [jnp.zeros((2, -s), jnp.float32), ut[:, :s]], axis=1)
            else:
                sh = ut
            if dx == 0:
                sh = jnp.where(wcol != 0, sh, 0.0)
            elif dx == 2:
                sh = jnp.where(wcol != W - 1, sh, 0.0)
            off = off + sh
        pos = lax.broadcasted_iota(jnp.int32, (1, HW), 1)
        gx = (pos // W).astype(jnp.float32)
        gy = (pos % W).astype(jnp.float32)
        px = jnp.clip(gx + off[0:1, :] + bc_ref[0], 0.0, float(H - 2))
        py = jnp.clip(gy + off[1:2, :] + bc_ref[1], 0.0, float(W - 2))
        qx = jnp.floor(px)
        qy = jnp.floor(py)
        fx = px - qx
        fy = py - qy
        base = b * HW + qx.astype(jnp.int32) * W + qy.astype(jnp.int32)
        idx_ref[0] = jnp.concatenate(
            [base, base + (W + 1), base + 1, base + W], axis=0)
        omx = 1.0 - fx
        omy = 1.0 - fy
        g_ref[0] = jnp.concatenate(
            [omx * omy, fx * fy, omx * fy, fx * omy], axis=0)


def _tc_index(x_flat, wflat, bc):
    return pl.pallas_call(
        _tc_index_kernel,
        grid=(B, HW // HWB),
        in_specs=[
            pl.BlockSpec((18, C), lambda b, i: (0, 0)),
            pl.BlockSpec((1, C, HWB), lambda b, i: (b, 0, i)),
            pl.BlockSpec(memory_space=pltpu.SMEM),
        ],
        out_specs=[
            pl.BlockSpec((1, 4, HW), lambda b, i: (b, 0, 0)),
            pl.BlockSpec((1, 4, HW), lambda b, i: (b, 0, 0)),
            pl.BlockSpec((HWB, C), lambda b, i: (b * (HW // HWB) + i, 0)),
        ],
        out_shape=[
            jax.ShapeDtypeStruct((B, 4, HW), jnp.int32),
            jax.ShapeDtypeStruct((B, 4, HW), jnp.float32),
            jax.ShapeDtypeStruct((NROWS, C), jnp.float32),
        ],
        scratch_shapes=[pltpu.VMEM((18, HW), jnp.float32)],
    )(wflat, x_flat, bc)


def _sc_gather_combine(xt, idx, g):
    mesh = plsc.VectorSubcoreMesh(core_axis_name="c", subcore_axis_name="s")

    @functools.partial(
        pl.kernel,
        mesh=mesh,
        out_type=jax.ShapeDtypeStruct((NROWS, C), jnp.float32),
        compiler_params=pltpu.CompilerParams(use_tc_tiling_on_sc=False),
        scratch_types=[
            pltpu.VMEM((RPW,), jnp.float32),
            pltpu.VMEM((RPW,), jnp.float32),
            pltpu.VMEM((RPW,), jnp.float32),
            pltpu.VMEM((RPW,), jnp.float32),
            pltpu.VMEM((CHUNK,), jnp.int32),
            pltpu.VMEM((CHUNK,), jnp.int32),
            pltpu.VMEM((CHUNK,), jnp.int32),
            pltpu.VMEM((CHUNK,), jnp.int32),
            pltpu.VMEM((CHUNK, C), jnp.float32),
            pltpu.VMEM((CHUNK, C), jnp.float32),
            pltpu.VMEM((CHUNK, C), jnp.float32),
            pltpu.VMEM((CHUNK, C), jnp.float32),
            pltpu.VMEM((CHUNK, C), jnp.float32),
            pltpu.SemaphoreType.DMA,
        ],
    )
    def sc_kernel(idx_hbm, g_hbm, xt_hbm, out_hbm,
                  gw0, gw1, gw2, gw3, ic0, ic1, ic2, ic3,
                  b0, b1, b2, b3, ob, sem):
        wid = lax.axis_index("s") * 2 + lax.axis_index("c")
        batch = wid // (HW // RPW)
        inb = (wid % (HW // RPW)) * RPW
        gws = (gw0, gw1, gw2, gw3)
        ics = (ic0, ic1, ic2, ic3)
        for k in range(4):
            pltpu.sync_copy(g_hbm.at[pl.ds((batch * 4 + k) * HW + inb, RPW)],
                            gws[k])

        @pl.loop(0, NCHUNK)
        def _chunk(j):
            off = j * CHUNK
            for k in range(4):
                pltpu.sync_copy(
                    idx_hbm.at[pl.ds((batch * 4 + k) * HW + inb + off, CHUNK)],
                    ics[k])
            bufs = (b0, b1, b2, b3)
            copies = [
                pltpu.async_copy(xt_hbm.at[ics[k]], bufs[k], sem)
                for k in range(4)
            ]
            for cp in copies:
                cp.wait()

            @pl.loop(0, CHUNK, step=16)
            def _grp(r0):
                gv0 = gw0[pl.ds(off + r0, 16)]
                gv1 = gw1[pl.ds(off + r0, 16)]
                gv2 = gw2[pl.ds(off + r0, 16)]
                gv3 = gw3[pl.ds(off + r0, 16)]
                for rr in range(16):
                    r = r0 + rr
                    g0, g1, g2, g3 = gv0[rr], gv1[rr], gv2[rr], gv3[rr]
                    for v in range(NV):
                        sl = pl.ds(v * 16, 16)
                        ob[r, sl] = (g0 * b0[r, sl] + g1 * b1[r, sl]
                                     + g2 * b2[r, sl] + g3 * b3[r, sl])

            pltpu.sync_copy(ob, out_hbm.at[pl.ds(wid * RPW + off, CHUNK)])

    return sc_kernel(idx.reshape(-1), g.reshape(-1), xt)


def _tc_transpose_out_kernel(in_ref, out_ref):
    out_ref[0] = in_ref[...].T


def _tc_transpose_out(rows):
    TB = HW // 8
    return pl.pallas_call(
        _tc_transpose_out_kernel,
        grid=(NROWS // TB,),
        in_specs=[pl.BlockSpec((TB, C), lambda i: (i, 0))],
        out_specs=pl.BlockSpec((1, C, TB), lambda i: (i // 8, 0, i % 8)),
        out_shape=jax.ShapeDtypeStruct((B, C, HW), jnp.float32),
    )(rows)


def kernel(x, Wc, bc):
    x_flat = x.reshape(B, C, HW)
    wflat = Wc.transpose(0, 2, 3, 1).reshape(2 * 9, C)
    idx, g, xt = _tc_index(x_flat, wflat, bc)
    out_rows = _sc_gather_combine(xt, idx, g)
    return _tc_transpose_out(out_rows).reshape(B, C, H, W, 1)


# speedup vs baseline: 6.9640x; 1.1555x over previous
import functools

import jax
import jax.numpy as jnp
from jax import lax
from jax.experimental import pallas as pl
from jax.experimental.pallas import tpu as pltpu
from jax.experimental.pallas import tpu_sc as plsc

B, C, H, W = 2, 192, 224, 224
HW = H * W
NROWS = B * HW
NW = 32
RPW = NROWS // NW
CHUNK = 64
NCHUNK = RPW // CHUNK
HWB = HW // 8
NV = C // 16


def _tc_index_kernel(wf_ref, x_ref, bc_ref, idx_ref, g_ref, xta_ref, xtb_ref,
                     u_ref):
    i = pl.program_id(1)
    xb = x_ref[0]
    u_ref[:, pl.ds(i * HWB, HWB)] = jnp.dot(
        wf_ref[...], xb, preferred_element_type=jnp.float32)
    xbt = xb.T
    xta_ref[...] = xbt[:, :128]
    xtb_ref[...] = jnp.concatenate(
        [xbt[:, 128:], jnp.zeros((HWB, 64), jnp.float32)], axis=1)

    @pl.when(i == (HW // HWB) - 1)
    def _():
        b = pl.program_id(0)
        u = u_ref[...]
        wcol = lax.broadcasted_iota(jnp.int32, (2, HW), 1) % W
        off = jnp.zeros((2, HW), jnp.float32)
        for t in range(9):
            dy, dx = t // 3, t % 3
            s = (dy - 1) * W + (dx - 1)
            ut = jnp.concatenate([u[t:t + 1, :], u[9 + t:10 + t, :]], axis=0)
            if s > 0:
                sh = jnp.concatenate(
                    [ut[:, s:], jnp.zeros((2, s), jnp.float32)], axis=1)
            elif s < 0:
                sh = jnp.concatenate(
                    [jnp.zeros((2, -s), jnp.float32), ut[:, :s]], axis=1)
            else:
                sh = ut
            if dx == 0:
                sh = jnp.where(wcol != 0, sh, 0.0)
            elif dx == 2:
                sh = jnp.where(wcol != W - 1, sh, 0.0)
            off = off + sh
        pos = lax.broadcasted_iota(jnp.int32, (1, HW), 1)
        gx = (pos // W).astype(jnp.float32)
        gy = (pos % W).astype(jnp.float32)
        px = jnp.clip(gx + off[0:1, :] + bc_ref[0], 0.0, float(H - 2))
        py = jnp.clip(gy + off[1:2, :] + bc_ref[1], 0.0, float(W - 2))
        qx = jnp.floor(px)
        qy = jnp.floor(py)
        fx = px - qx
        fy = py - qy
        base = b * HW + qx.astype(jnp.int32) * W + qy.astype(jnp.int32)
        idx_ref[0] = jnp.concatenate(
            [base, base + (W + 1), base + 1, base + W], axis=0)
        omx = 1.0 - fx
        omy = 1.0 - fy
        g_ref[0] = jnp.concatenate(
            [omx * omy, fx * fy, omx * fy, fx * omy], axis=0)


def _tc_index(x_flat, wflat, bc):
    return pl.pallas_call(
        _tc_index_kernel,
        grid=(B, HW // HWB),
        in_specs=[
            pl.BlockSpec((18, C), lambda b, i: (0, 0)),
            pl.BlockSpec((1, C, HWB), lambda b, i: (b, 0, i)),
            pl.BlockSpec(memory_space=pltpu.SMEM),
        ],
        out_specs=[
            pl.BlockSpec((1, 4, HW), lambda b, i: (b, 0, 0)),
            pl.BlockSpec((1, 4, HW), lambda b, i: (b, 0, 0)),
            pl.BlockSpec((HWB, 128), lambda b, i: (b * (HW // HWB) + i, 0)),
            pl.BlockSpec((HWB, 128), lambda b, i: (b * (HW // HWB) + i, 0)),
        ],
        out_shape=[
            jax.ShapeDtypeStruct((B, 4, HW), jnp.int32),
            jax.ShapeDtypeStruct((B, 4, HW), jnp.float32),
            jax.ShapeDtypeStruct((NROWS, 128), jnp.float32),
            jax.ShapeDtypeStruct((NROWS, 128), jnp.float32),
        ],
        scratch_shapes=[pltpu.VMEM((18, HW), jnp.float32)],
    )(wflat, x_flat, bc)


def _sc_gather_combine(xta, xtb, idx, g):
    mesh = plsc.VectorSubcoreMesh(core_axis_name="c", subcore_axis_name="s")

    @functools.partial(
        pl.kernel,
        mesh=mesh,
        out_type=[
            jax.ShapeDtypeStruct((NROWS, 128), jnp.float32),
            jax.ShapeDtypeStruct((NROWS, 128), jnp.float32),
        ],
        compiler_params=pltpu.CompilerParams(use_tc_tiling_on_sc=False),
        scratch_types=[
            pltpu.VMEM((RPW,), jnp.float32),
            pltpu.VMEM((RPW,), jnp.float32),
            pltpu.VMEM((RPW,), jnp.float32),
            pltpu.VMEM((RPW,), jnp.float32),
            pltpu.VMEM((CHUNK,), jnp.int32),
            pltpu.VMEM((CHUNK,), jnp.int32),
            pltpu.VMEM((CHUNK,), jnp.int32),
            pltpu.VMEM((CHUNK,), jnp.int32),
            pltpu.VMEM((CHUNK, 128), jnp.float32),
            pltpu.VMEM((CHUNK, 128), jnp.float32),
            pltpu.VMEM((CHUNK, 128), jnp.float32),
            pltpu.VMEM((CHUNK, 128), jnp.float32),
            pltpu.VMEM((CHUNK, 128), jnp.float32),
            pltpu.VMEM((CHUNK, 128), jnp.float32),
            pltpu.VMEM((CHUNK, 128), jnp.float32),
            pltpu.VMEM((CHUNK, 128), jnp.float32),
            pltpu.VMEM((CHUNK, 128), jnp.float32),
            pltpu.VMEM((CHUNK, 128), jnp.float32),
            pltpu.SemaphoreType.DMA,
        ],
    )
    def sc_kernel(idx_hbm, g_hbm, xta_hbm, xtb_hbm, outa_hbm, outb_hbm,
                  gw0, gw1, gw2, gw3, ic0, ic1, ic2, ic3,
                  a0, a1, a2, a3, e0, e1, e2, e3, oa, ob_, sem):
        wid = lax.axis_index("s") * 2 + lax.axis_index("c")
        batch = wid // (HW // RPW)
        inb = (wid % (HW // RPW)) * RPW
        gws = (gw0, gw1, gw2, gw3)
        ics = (ic0, ic1, ic2, ic3)
        for k in range(4):
            pltpu.sync_copy(g_hbm.at[pl.ds((batch * 4 + k) * HW + inb, RPW)],
                            gws[k])

        @pl.loop(0, NCHUNK)
        def _chunk(j):
            off = j * CHUNK
            for k in range(4):
                pltpu.sync_copy(
                    idx_hbm.at[pl.ds((batch * 4 + k) * HW + inb + off, CHUNK)],
                    ics[k])
            abufs = (a0, a1, a2, a3)
            bbufs = (e0, e1, e2, e3)
            copies = [
                pltpu.async_copy(xta_hbm.at[ics[k]], abufs[k], sem)
                for k in range(4)
            ] + [
                pltpu.async_copy(xtb_hbm.at[ics[k]], bbufs[k], sem)
                for k in range(4)
            ]
            for cp in copies:
                cp.wait()

            @pl.loop(0, CHUNK, step=16)
            def _grp(r0):
                gv0 = gw0[pl.ds(off + r0, 16)]
                gv1 = gw1[pl.ds(off + r0, 16)]
                gv2 = gw2[pl.ds(off + r0, 16)]
                gv3 = gw3[pl.ds(off + r0, 16)]
                for rr in range(16):
                    r = r0 + rr
                    g0, g1, g2, g3 = gv0[rr], gv1[rr], gv2[rr], gv3[rr]
                    for v in range(8):
                        sl = pl.ds(v * 16, 16)
                        oa[r, sl] = (g0 * a0[r, sl] + g1 * a1[r, sl]
                                     + g2 * a2[r, sl] + g3 * a3[r, sl])
                    for v in range(4):
                        sl = pl.ds(v * 16, 16)
                        ob_[r, sl] = (g0 * e0[r, sl] + g1 * e1[r, sl]
                                      + g2 * e2[r, sl] + g3 * e3[r, sl])

            pltpu.sync_copy(oa, outa_hbm.at[pl.ds(wid * RPW + off, CHUNK)])
            pltpu.sync_copy(ob_, outb_hbm.at[pl.ds(wid * RPW + off, CHUNK)])

    return sc_kernel(idx.reshape(-1), g.reshape(-1), xta, xtb)


def _tc_transpose_out_kernel(ina_ref, inb_ref, out_ref):
    out_ref[0] = jnp.concatenate(
        [ina_ref[...].T, inb_ref[:, :64].T], axis=0)


def _tc_transpose_out(rows_a, rows_b):
    TB = HW // 8
    return pl.pallas_call(
        _tc_transpose_out_kernel,
        grid=(NROWS // TB,),
        in_specs=[
            pl.BlockSpec((TB, 128), lambda i: (i, 0)),
            pl.BlockSpec((TB, 128), lambda i: (i, 0)),
        ],
        out_specs=pl.BlockSpec((1, C, TB), lambda i: (i // 8, 0, i % 8)),
        out_shape=jax.ShapeDtypeStruct((B, C, HW), jnp.float32),
    )(rows_a, rows_b)


def kernel(x, Wc, bc):
    x_flat = x.reshape(B, C, HW)
    wflat = Wc.transpose(0, 2, 3, 1).reshape(2 * 9, C)
    idx, g, xta, xtb = _tc_index(x_flat, wflat, bc)
    out_a, out_b = _sc_gather_combine(xta, xtb, idx, g)
    return _tc_transpose_out(out_a, out_b).reshape(B, C, H, W, 1)


# speedup vs baseline: 9.8926x; 1.4205x over previous
import functools

import jax
import jax.numpy as jnp
from jax import lax
from jax.experimental import pallas as pl
from jax.experimental.pallas import tpu as pltpu
from jax.experimental.pallas import tpu_sc as plsc

B, C, H, W = 2, 192, 224, 224
HW = H * W
NROWS = B * HW
NW = 32
RPW = NROWS // NW
CHUNK = 64
NCHUNK = RPW // CHUNK
HB = 32
HWB = HB * W
NV = C // 16


def _tc_index_kernel(wf_ref, x_ref, bc_ref, idx_ref, g_ref, xta_ref, xtb_ref,
                     u_ref):
    i = pl.program_id(1)
    xb = x_ref[0].reshape(C, HWB)
    u_ref[:, pl.ds(i * HWB, HWB)] = jnp.dot(
        wf_ref[...], xb, preferred_element_type=jnp.float32)
    xbt = xb.T
    xta_ref[...] = xbt[:, :128]
    xtb_ref[...] = jnp.concatenate(
        [xbt[:, 128:], jnp.zeros((HWB, 64), jnp.float32)], axis=1)

    @pl.when(i == (HW // HWB) - 1)
    def _():
        b = pl.program_id(0)
        u = u_ref[...]
        wcol = lax.broadcasted_iota(jnp.int32, (2, HW), 1) % W
        off = jnp.zeros((2, HW), jnp.float32)
        for t in range(9):
            dy, dx = t // 3, t % 3
            s = (dy - 1) * W + (dx - 1)
            ut = jnp.concatenate([u[t:t + 1, :], u[9 + t:10 + t, :]], axis=0)
            if s > 0:
                sh = jnp.concatenate(
                    [ut[:, s:], jnp.zeros((2, s), jnp.float32)], axis=1)
            elif s < 0:
                sh = jnp.concatenate(
                    [jnp.zeros((2, -s), jnp.float32), ut[:, :s]], axis=1)
            else:
                sh = ut
            if dx == 0:
                sh = jnp.where(wcol != 0, sh, 0.0)
            elif dx == 2:
                sh = jnp.where(wcol != W - 1, sh, 0.0)
            off = off + sh
        pos = lax.broadcasted_iota(jnp.int32, (1, HW), 1)
        gx = (pos // W).astype(jnp.float32)
        gy = (pos % W).astype(jnp.float32)
        px = jnp.clip(gx + off[0:1, :] + bc_ref[0], 0.0, float(H - 2))
        py = jnp.clip(gy + off[1:2, :] + bc_ref[1], 0.0, float(W - 2))
        qx = jnp.floor(px)
        qy = jnp.floor(py)
        fx = px - qx
        fy = py - qy
        base = b * HW + qx.astype(jnp.int32) * W + qy.astype(jnp.int32)
        idx_ref[0] = jnp.concatenate(
            [base, base + (W + 1), base + 1, base + W], axis=0)
        omx = 1.0 - fx
        omy = 1.0 - fy
        g_ref[0] = jnp.concatenate(
            [omx * omy, fx * fy, omx * fy, fx * omy], axis=0)


def _tc_index(x4, wflat, bc):
    return pl.pallas_call(
        _tc_index_kernel,
        grid=(B, HW // HWB),
        in_specs=[
            pl.BlockSpec((18, C), lambda b, i: (0, 0)),
            pl.BlockSpec((1, C, HB, W), lambda b, i: (b, 0, i, 0)),
            pl.BlockSpec(memory_space=pltpu.SMEM),
        ],
        out_specs=[
            pl.BlockSpec((1, 4, HW), lambda b, i: (b, 0, 0)),
            pl.BlockSpec((1, 4, HW), lambda b, i: (b, 0, 0)),
            pl.BlockSpec((HWB, 128), lambda b, i: (b * (HW // HWB) + i, 0)),
            pl.BlockSpec((HWB, 128), lambda b, i: (b * (HW // HWB) + i, 0)),
        ],
        out_shape=[
            jax.ShapeDtypeStruct((B, 4, HW), jnp.int32),
            jax.ShapeDtypeStruct((B, 4, HW), jnp.float32),
            jax.ShapeDtypeStruct((NROWS, 128), jnp.float32),
            jax.ShapeDtypeStruct((NROWS, 128), jnp.float32),
        ],
        scratch_shapes=[pltpu.VMEM((18, HW), jnp.float32)],
    )(wflat, x4, bc)


def _sc_gather_combine(xta, xtb, idx, g):
    mesh = plsc.VectorSubcoreMesh(core_axis_name="c", subcore_axis_name="s")

    @functools.partial(
        pl.kernel,
        mesh=mesh,
        out_type=[
            jax.ShapeDtypeStruct((NROWS, 128), jnp.float32),
            jax.ShapeDtypeStruct((NROWS, 128), jnp.float32),
        ],
        compiler_params=pltpu.CompilerParams(use_tc_tiling_on_sc=False),
        scratch_types=[
            pltpu.VMEM((RPW,), jnp.float32),
            pltpu.VMEM((RPW,), jnp.float32),
            pltpu.VMEM((RPW,), jnp.float32),
            pltpu.VMEM((RPW,), jnp.float32),
            pltpu.VMEM((CHUNK,), jnp.int32),
            pltpu.VMEM((CHUNK,), jnp.int32),
            pltpu.VMEM((CHUNK,), jnp.int32),
            pltpu.VMEM((CHUNK,), jnp.int32),
            pltpu.VMEM((CHUNK, 128), jnp.float32),
            pltpu.VMEM((CHUNK, 128), jnp.float32),
            pltpu.VMEM((CHUNK, 128), jnp.float32),
            pltpu.VMEM((CHUNK, 128), jnp.float32),
            pltpu.VMEM((CHUNK, 128), jnp.float32),
            pltpu.VMEM((CHUNK, 128), jnp.float32),
            pltpu.VMEM((CHUNK, 128), jnp.float32),
            pltpu.VMEM((CHUNK, 128), jnp.float32),
            pltpu.VMEM((CHUNK, 128), jnp.float32),
            pltpu.VMEM((CHUNK, 128), jnp.float32),
            pltpu.SemaphoreType.DMA,
        ],
    )
    def sc_kernel(idx_hbm, g_hbm, xta_hbm, xtb_hbm, outa_hbm, outb_hbm,
                  gw0, gw1, gw2, gw3, ic0, ic1, ic2, ic3,
                  a0, a1, a2, a3, e0, e1, e2, e3, oa, ob_, sem):
        wid = lax.axis_index("s") * 2 + lax.axis_index("c")
        batch = wid // (HW // RPW)
        inb = (wid % (HW // RPW)) * RPW
        gws = (gw0, gw1, gw2, gw3)
        ics = (ic0, ic1, ic2, ic3)
        for k in range(4):
            pltpu.sync_copy(g_hbm.at[pl.ds((batch * 4 + k) * HW + inb, RPW)],
                            gws[k])

        @pl.loop(0, NCHUNK)
        def _chunk(j):
            off = j * CHUNK
            for k in range(4):
                pltpu.sync_copy(
                    idx_hbm.at[pl.ds((batch * 4 + k) * HW + inb + off, CHUNK)],
                    ics[k])
            abufs = (a0, a1, a2, a3)
            bbufs = (e0, e1, e2, e3)
            copies = [
                pltpu.async_copy(xta_hbm.at[ics[k]], abufs[k], sem)
                for k in range(4)
            ] + [
                pltpu.async_copy(xtb_hbm.at[ics[k]], bbufs[k], sem)
                for k in range(4)
            ]
            for cp in copies:
                cp.wait()

            @pl.loop(0, CHUNK, step=16)
            def _grp(r0):
                gv0 = gw0[pl.ds(off + r0, 16)]
                gv1 = gw1[pl.ds(off + r0, 16)]
                gv2 = gw2[pl.ds(off + r0, 16)]
                gv3 = gw3[pl.ds(off + r0, 16)]
                for rr in range(16):
                    r = r0 + rr
                    g0, g1, g2, g3 = gv0[rr], gv1[rr], gv2[rr], gv3[rr]
                    for v in range(8):
                        sl = pl.ds(v * 16, 16)
                        oa[r, sl] = (g0 * a0[r, sl] + g1 * a1[r, sl]
                                     + g2 * a2[r, sl] + g3 * a3[r, sl])
                    for v in range(4):
                        sl = pl.ds(v * 16, 16)
                        ob_[r, sl] = (g0 * e0[r, sl] + g1 * e1[r, sl]
                                      + g2 * e2[r, sl] + g3 * e3[r, sl])

            pltpu.sync_copy(oa, outa_hbm.at[pl.ds(wid * RPW + off, CHUNK)])
            pltpu.sync_copy(ob_, outb_hbm.at[pl.ds(wid * RPW + off, CHUNK)])

    return sc_kernel(idx.reshape(-1), g.reshape(-1), xta, xtb)


def _tc_transpose_out_kernel(ina_ref, inb_ref, out_ref):
    ct = jnp.concatenate([ina_ref[...].T, inb_ref[:, :64].T], axis=0)
    out_ref[...] = ct.reshape(C, HB, W)[None]


def _tc_transpose_out(rows_a, rows_b):
    TB = HB * W
    return pl.pallas_call(
        _tc_transpose_out_kernel,
        grid=(NROWS // TB,),
        in_specs=[
            pl.BlockSpec((TB, 128), lambda i: (i, 0)),
            pl.BlockSpec((TB, 128), lambda i: (i, 0)),
        ],
        out_specs=pl.BlockSpec((1, C, HB, W),
                               lambda i: (i // (H // HB), 0, i % (H // HB), 0)),
        out_shape=jax.ShapeDtypeStruct((B, C, H, W), jnp.float32),
    )(rows_a, rows_b)


def kernel(x, Wc, bc):
    wflat = Wc.transpose(0, 2, 3, 1).reshape(2 * 9, C)
    idx, g, xta, xtb = _tc_index(x, wflat, bc)
    out_a, out_b = _sc_gather_combine(xta, xtb, idx, g)
    return _tc_transpose_out(out_a, out_b).reshape(B, C, H, W, 1)


# speedup vs baseline: 11.4388x; 1.1563x over previous
import functools

import jax
import jax.numpy as jnp
from jax import lax
from jax.experimental import pallas as pl
from jax.experimental.pallas import tpu as pltpu
from jax.experimental.pallas import tpu_sc as plsc

B, C, H, W = 2, 192, 224, 224
HW = H * W
NROWS = B * HW
NW = 32
RPW = NROWS // NW
CHUNK = 64
NCHUNK = RPW // CHUNK
HB = 32
HWB = HB * W
NV = C // 16


def _tc_index_kernel(wf_ref, x_ref, xn_ref, bc_ref, idx_ref, g_ref, xta_ref,
                     xtb_ref, u_ref):
    i = pl.program_id(1)
    xb = x_ref[0].reshape(C, HWB)
    u_ref[:, pl.ds(i * HWB, HWB)] = jnp.dot(
        wf_ref[...], xb, preferred_element_type=jnp.float32)
    xbt = xb.T
    xta_ref[...] = xbt[:, :128]
    bpart = xbt[:, 128:]
    hb0 = xn_ref[0, 128:, 0, 0:1].T
    nxt = jnp.concatenate([bpart, hb0], axis=0)[1:]
    xtb_ref[...] = jnp.concatenate([bpart, nxt], axis=1)

    @pl.when(i == (HW // HWB) - 1)
    def _():
        b = pl.program_id(0)
        u = u_ref[...]
        wcol = lax.broadcasted_iota(jnp.int32, (2, HW), 1) % W
        off = jnp.zeros((2, HW), jnp.float32)
        for t in range(9):
            dy, dx = t // 3, t % 3
            s = (dy - 1) * W + (dx - 1)
            ut = jnp.concatenate([u[t:t + 1, :], u[9 + t:10 + t, :]], axis=0)
            if s > 0:
                sh = jnp.concatenate(
                    [ut[:, s:], jnp.zeros((2, s), jnp.float32)], axis=1)
            elif s < 0:
                sh = jnp.concatenate(
                    [jnp.zeros((2, -s), jnp.float32), ut[:, :s]], axis=1)
            else:
                sh = ut
            if dx == 0:
                sh = jnp.where(wcol != 0, sh, 0.0)
            elif dx == 2:
                sh = jnp.where(wcol != W - 1, sh, 0.0)
            off = off + sh
        pos = lax.broadcasted_iota(jnp.int32, (1, HW), 1)
        gx = (pos // W).astype(jnp.float32)
        gy = (pos % W).astype(jnp.float32)
        px = jnp.clip(gx + off[0:1, :] + bc_ref[0], 0.0, float(H - 2))
        py = jnp.clip(gy + off[1:2, :] + bc_ref[1], 0.0, float(W - 2))
        qx = jnp.floor(px)
        qy = jnp.floor(py)
        fx = px - qx
        fy = py - qy
        base = b * HW + qx.astype(jnp.int32) * W + qy.astype(jnp.int32)
        idx_ref[0] = jnp.concatenate(
            [base, base + (W + 1), base + 1, base + W], axis=0)
        omx = 1.0 - fx
        omy = 1.0 - fy
        g_ref[0] = jnp.concatenate(
            [omx * omy, fx * fy, omx * fy, fx * omy], axis=0)


def _tc_index(x4, wflat, bc):
    return pl.pallas_call(
        _tc_index_kernel,
        grid=(B, HW // HWB),
        in_specs=[
            pl.BlockSpec((18, C), lambda b, i: (0, 0)),
            pl.BlockSpec((1, C, HB, W), lambda b, i: (b, 0, i, 0)),
            pl.BlockSpec((1, C, 8, W),
                         lambda b, i: (b, 0,
                                       jnp.minimum((i + 1) * (HB // 8),
                                                   H // 8 - 1), 0)),
            pl.BlockSpec(memory_space=pltpu.SMEM),
        ],
        out_specs=[
            pl.BlockSpec((1, 4, HW), lambda b, i: (b, 0, 0)),
            pl.BlockSpec((1, 4, HW), lambda b, i: (b, 0, 0)),
            pl.BlockSpec((HWB, 128), lambda b, i: (b * (HW // HWB) + i, 0)),
            pl.BlockSpec((HWB, 128), lambda b, i: (b * (HW // HWB) + i, 0)),
        ],
        out_shape=[
            jax.ShapeDtypeStruct((B, 4, HW), jnp.int32),
            jax.ShapeDtypeStruct((B, 4, HW), jnp.float32),
            jax.ShapeDtypeStruct((NROWS, 128), jnp.float32),
            jax.ShapeDtypeStruct((NROWS, 128), jnp.float32),
        ],
        scratch_shapes=[pltpu.VMEM((18, HW), jnp.float32)],
    )(wflat, x4, x4, bc)


def _sc_gather_combine(xta, xtb, idx, g):
    mesh = plsc.VectorSubcoreMesh(core_axis_name="c", subcore_axis_name="s")

    @functools.partial(
        pl.kernel,
        mesh=mesh,
        out_type=[
            jax.ShapeDtypeStruct((NROWS, 128), jnp.float32),
            jax.ShapeDtypeStruct((NROWS, 128), jnp.float32),
        ],
        compiler_params=pltpu.CompilerParams(use_tc_tiling_on_sc=False),
        scratch_types=[
            pltpu.VMEM((RPW,), jnp.float32),
            pltpu.VMEM((RPW,), jnp.float32),
            pltpu.VMEM((RPW,), jnp.float32),
            pltpu.VMEM((RPW,), jnp.float32),
            pltpu.VMEM((RPW,), jnp.int32),
            pltpu.VMEM((RPW,), jnp.int32),
            pltpu.VMEM((RPW,), jnp.int32),
            pltpu.VMEM((RPW,), jnp.int32),
            pltpu.VMEM((CHUNK, 128), jnp.float32),
            pltpu.VMEM((CHUNK, 128), jnp.float32),
            pltpu.VMEM((CHUNK, 128), jnp.float32),
            pltpu.VMEM((CHUNK, 128), jnp.float32),
            pltpu.VMEM((CHUNK, 128), jnp.float32),
            pltpu.VMEM((CHUNK, 128), jnp.float32),
            pltpu.VMEM((CHUNK, 128), jnp.float32),
            pltpu.VMEM((CHUNK, 128), jnp.float32),
            pltpu.SemaphoreType.DMA,
        ],
    )
    def sc_kernel(idx_hbm, g_hbm, xta_hbm, xtb_hbm, outa_hbm, outb_hbm,
                  gw0, gw1, gw2, gw3, iv0, iv1, iv2, iv3,
                  a0, a1, a2, a3, p0, p1, oa, ob_, sem):
        wid = lax.axis_index("s") * 2 + lax.axis_index("c")
        batch = wid // (HW // RPW)
        inb = (wid % (HW // RPW)) * RPW
        gws = (gw0, gw1, gw2, gw3)
        ivs = (iv0, iv1, iv2, iv3)
        for k in range(4):
            pltpu.sync_copy(g_hbm.at[pl.ds((batch * 4 + k) * HW + inb, RPW)],
                            gws[k])
            pltpu.sync_copy(idx_hbm.at[pl.ds((batch * 4 + k) * HW + inb, RPW)],
                            ivs[k])

        @pl.loop(0, NCHUNK)
        def _chunk(j):
            off = j * CHUNK
            abufs = (a0, a1, a2, a3)
            copies = [
                pltpu.async_copy(xta_hbm.at[ivs[k].at[pl.ds(off, CHUNK)]],
                                 abufs[k], sem)
                for k in range(4)
            ] + [
                pltpu.async_copy(xtb_hbm.at[iv0.at[pl.ds(off, CHUNK)]],
                                 p0, sem),
                pltpu.async_copy(xtb_hbm.at[iv3.at[pl.ds(off, CHUNK)]],
                                 p1, sem),
            ]
            for cp in copies:
                cp.wait()

            @pl.loop(0, CHUNK, step=16)
            def _grp(r0):
                gv0 = gw0[pl.ds(off + r0, 16)]
                gv1 = gw1[pl.ds(off + r0, 16)]
                gv2 = gw2[pl.ds(off + r0, 16)]
                gv3 = gw3[pl.ds(off + r0, 16)]
                for rr in range(16):
                    r = r0 + rr
                    g0, g1, g2, g3 = gv0[rr], gv1[rr], gv2[rr], gv3[rr]
                    for v in range(8):
                        sl = pl.ds(v * 16, 16)
                        oa[r, sl] = (g0 * a0[r, sl] + g1 * a1[r, sl]
                                     + g2 * a2[r, sl] + g3 * a3[r, sl])
                    for v in range(4):
                        sl = pl.ds(v * 16, 16)
                        sh = pl.ds(64 + v * 16, 16)
                        ob_[r, sl] = (g0 * p0[r, sl] + g2 * p0[r, sh]
                                      + g3 * p1[r, sl] + g1 * p1[r, sh])

            pltpu.sync_copy(oa, outa_hbm.at[pl.ds(wid * RPW + off, CHUNK)])
            pltpu.sync_copy(ob_, outb_hbm.at[pl.ds(wid * RPW + off, CHUNK)])

    return sc_kernel(idx.reshape(-1), g.reshape(-1), xta, xtb)


def _tc_transpose_out_kernel(ina_ref, inb_ref, out_ref):
    ct = jnp.concatenate([ina_ref[...].T, inb_ref[:, :64].T], axis=0)
    out_ref[...] = ct.reshape(C, HB, W)[None]


def _tc_transpose_out(rows_a, rows_b):
    TB = HB * W
    return pl.pallas_call(
        _tc_transpose_out_kernel,
        grid=(NROWS // TB,),
        in_specs=[
            pl.BlockSpec((TB, 128), lambda i: (i, 0)),
            pl.BlockSpec((TB, 128), lambda i: (i, 0)),
        ],
        out_specs=pl.BlockSpec((1, C, HB, W),
                               lambda i: (i // (H // HB), 0, i % (H // HB), 0)),
        out_shape=jax.ShapeDtypeStruct((B, C, H, W), jnp.float32),
    )(rows_a, rows_b)


def kernel(x, Wc, bc):
    wflat = Wc.transpose(0, 2, 3, 1).reshape(2 * 9, C)
    idx, g, xta, xtb = _tc_index(x, wflat, bc)
    out_a, out_b = _sc_gather_combine(xta, xtb, idx, g)
    return _tc_transpose_out(out_a, out_b).reshape(B, C, H, W, 1)
